# Initial kernel scaffold; baseline (speedup 1.0000x reference)
import jax
import jax.numpy as jnp
from jax.experimental import pallas as pl


def kernel(t_span, dt, A_real, A_imag, w_acc_real, w_acc_imag, theta, W_filter_real, W_filter_imag, L_real_init, L_imag_init):
    raise NotImplementedError("write your pallas kernel here")


# speedup vs baseline: 1.5907x; 1.5907x over previous
import functools

import jax
import jax.numpy as jnp
from jax import lax
from jax.experimental import pallas as pl
from jax.experimental.pallas import tpu as pltpu
from jax.experimental.pallas import tpu_sc as plsc

N = 4096
LANES = 16
BATCH = 8
NCHUNK = N // LANES


def _prep_body(dt_ref, ar, ai, wr, wi, th, lr0, li0, olr, oli, oamt):
    dtf = dt_ref[0, 0]
    er = jnp.exp(ar[...] * dtf)
    exp_r = er * jnp.cos(ai[...] * dtf)
    exp_i = er * jnp.sin(ai[...] * dtf)
    lr = lr0[...] * exp_r - li0[...] * exp_i
    li = lr0[...] * exp_i + li0[...] * exp_r
    olr[...] = lr
    oli[...] = li
    oamt[...] = wr[...] * lr + wi[...] * li - th[...]


def _prep(dtf, ar, ai, wr, wi, th, lr0, li0, interpret=False):
    shp = (N // 128, 128)
    vspec = pl.BlockSpec(memory_space=pltpu.VMEM)
    outs = pl.pallas_call(
        _prep_body,
        out_shape=[jax.ShapeDtypeStruct(shp, jnp.float32)] * 3,
        in_specs=[pl.BlockSpec(memory_space=pltpu.SMEM)] + [vspec] * 7,
        out_specs=[vspec] * 3,
        interpret=interpret,
    )(dtf.reshape(1, 1), ar.reshape(shp), ai.reshape(shp), wr.reshape(shp),
      wi.reshape(shp), th.reshape(shp), lr0.reshape(shp), li0.reshape(shp))
    return tuple(o.reshape(N) for o in outs)


def _build_collapse(interpret=False, num_cores=None, num_subcores=None):
    if num_cores is None:
        mesh = plsc.VectorSubcoreMesh(core_axis_name="c", subcore_axis_name="s")
    else:
        mesh = plsc.VectorSubcoreMesh(core_axis_name="c", subcore_axis_name="s",
                                      num_cores=num_cores,
                                      num_subcores=num_subcores)
    nc, ns = mesh.num_cores, mesh.num_subcores
    nw = nc * ns
    rpt = N // nw

    @functools.partial(
        pl.kernel,
        out_type=(jax.ShapeDtypeStruct((N,), jnp.float32),
                  jax.ShapeDtypeStruct((N,), jnp.float32)),
        mesh=mesh,
        interpret=interpret,
        compiler_params=pltpu.CompilerParams(needs_layout_passes=False),
        scratch_types=[
            pltpu.VMEM((N,), jnp.float32),
            pltpu.VMEM((N,), jnp.float32),
            pltpu.VMEM((rpt,), jnp.float32),
            pltpu.VMEM((rpt,), jnp.int32),
            pltpu.VMEM((BATCH, N), jnp.float32),
            pltpu.VMEM((BATCH, N), jnp.float32),
            pltpu.VMEM((rpt,), jnp.float32),
            pltpu.VMEM((rpt,), jnp.float32),
            pltpu.SemaphoreType.DMA,
            pltpu.SemaphoreType.DMA,
        ],
    )
    def collapse(wr_hbm, wi_hbm, lr_hbm, li_hbm, amt_hbm,
                 outr_hbm, outi_hbm,
                 lr_v, li_v, amt_v, idx_v, rowr_v, rowi_v,
                 outr_v, outi_v, semr, semi):
        wid = lax.axis_index("s") * nc + lax.axis_index("c")
        row0 = wid * rpt

        pltpu.sync_copy(lr_hbm, lr_v)
        pltpu.sync_copy(li_hbm, li_v)
        pltpu.sync_copy(amt_hbm.at[pl.ds(row0, rpt)], amt_v)
        pltpu.sync_copy(lr_hbm.at[pl.ds(row0, rpt)], outr_v)
        pltpu.sync_copy(li_hbm.at[pl.ds(row0, rpt)], outi_v)

        iota = lax.broadcasted_iota(jnp.int32, (LANES,), 0)
        row0_v = jnp.full((LANES,), row0, jnp.int32)

        for k in range(rpt // LANES):
            idx_v[pl.ds(k * LANES, LANES)] = row0_v

        cnt = jnp.int32(0)
        for k in range(rpt // LANES):
            m = amt_v[pl.ds(k * LANES, LANES)] >= 0.0
            mi = m.astype(jnp.int32)
            pos = cnt + jnp.cumsum(mi) - 1
            plsc.store_scatter(idx_v, [pos], row0_v + (k * LANES) + iota,
                               mask=m)
            cnt = cnt + jnp.sum(mi)

        nb = (cnt + (BATCH - 1)) // BATCH
        zeros = jnp.zeros((LANES,), jnp.float32)
        cnt_v = jnp.full((LANES,), cnt, jnp.int32)

        def batch_body(b, carry):
            sl = idx_v.at[pl.ds(b * BATCH, BATCH)]
            cp_r = pltpu.async_copy(wr_hbm.at[sl], rowr_v, semr)
            cp_i = pltpu.async_copy(wi_hbm.at[sl], rowi_v, semi)
            cp_r.wait()
            cp_i.wait()

            def chunk_body(c, accs):
                fr_t, fi_t = accs
                base = c * LANES
                lr_c = lr_v[pl.ds(base, LANES)]
                li_c = li_v[pl.ds(base, LANES)]
                nfr, nfi = [], []
                for r in range(BATCH):
                    w_r = rowr_v[r, pl.ds(base, LANES)]
                    w_i = rowi_v[r, pl.ds(base, LANES)]
                    nfr.append(fr_t[r] + (w_r * lr_c - w_i * li_c))
                    nfi.append(fi_t[r] + (w_r * li_c + w_i * lr_c))
                return (tuple(nfr), tuple(nfi))

            init = (tuple([zeros] * BATCH), tuple([zeros] * BATCH))
            fr_t, fi_t = lax.fori_loop(0, NCHUNK, chunk_body, init)

            for r in range(BATCH):
                slot = b * BATCH + r
                slot_v = jnp.full((LANES,), slot, jnp.int32)
                gidx_v = plsc.load_gather(idx_v, [slot_v])
                lidx_v = gidx_v - row0_v
                lrv = plsc.load_gather(lr_v, [gidx_v])
                liv = plsc.load_gather(li_v, [gidx_v])
                fr_v = jnp.full((LANES,), jnp.sum(fr_t[r]))
                fi_v = jnp.full((LANES,), jnp.sum(fi_t[r]))
                new_r = lrv * jnp.maximum(fr_v, 0.0)
                new_i = liv * fi_v
                mk = (iota == 0) & (slot_v < cnt_v)
                plsc.store_scatter(outr_v, [lidx_v], new_r, mask=mk)
                plsc.store_scatter(outi_v, [lidx_v], new_i, mask=mk)
            return carry

        lax.fori_loop(0, nb, batch_body, jnp.int32(0))

        pltpu.sync_copy(outr_v, outr_hbm.at[pl.ds(row0, rpt)])
        pltpu.sync_copy(outi_v, outi_hbm.at[pl.ds(row0, rpt)])

    return collapse


def kernel(t_span, dt, A_real, A_imag, w_acc_real, w_acc_imag, theta,
           W_filter_real, W_filter_imag, L_real_init, L_imag_init):
    num_steps = t_span.shape[0] - 1
    dtf = jnp.asarray(dt, jnp.float32)
    collapse = _build_collapse()
    Lr, Li = L_real_init, L_imag_init
    reals, imags = [], []
    for _ in range(num_steps):
        lr1, li1, amt = _prep(dtf, A_real, A_imag, w_acc_real, w_acc_imag,
                              theta, Lr, Li)
        Lr, Li = collapse(W_filter_real, W_filter_imag, lr1, li1, amt)
        reals.append(Lr)
        imags.append(Li)
    return jnp.stack(reals), jnp.stack(imags)


# speedup vs baseline: 1.9897x; 1.2509x over previous
import functools

import jax
import jax.numpy as jnp
from jax import lax
from jax.experimental import pallas as pl
from jax.experimental.pallas import tpu as pltpu
from jax.experimental.pallas import tpu_sc as plsc

N = 4096
LANES = 16
BATCH = 8
NCHUNK = N // LANES


def _prep_body(dt_ref, ar, ai, wr, wi, th, lr0, li0, olr, oli, oamt):
    dtf = dt_ref[0, 0]
    er = jnp.exp(ar[...] * dtf)
    exp_r = er * jnp.cos(ai[...] * dtf)
    exp_i = er * jnp.sin(ai[...] * dtf)
    lr = lr0[...] * exp_r - li0[...] * exp_i
    li = lr0[...] * exp_i + li0[...] * exp_r
    olr[...] = lr
    oli[...] = li
    oamt[...] = wr[...] * lr + wi[...] * li - th[...]


def _prep(dtf, ar, ai, wr, wi, th, lr0, li0, interpret=False):
    shp = (N // 128, 128)
    vspec = pl.BlockSpec(memory_space=pltpu.VMEM)
    outs = pl.pallas_call(
        _prep_body,
        out_shape=[jax.ShapeDtypeStruct(shp, jnp.float32)] * 3,
        in_specs=[pl.BlockSpec(memory_space=pltpu.SMEM)] + [vspec] * 7,
        out_specs=[vspec] * 3,
        interpret=interpret,
    )(dtf.reshape(1, 1), ar.reshape(shp), ai.reshape(shp), wr.reshape(shp),
      wi.reshape(shp), th.reshape(shp), lr0.reshape(shp), li0.reshape(shp))
    return tuple(o.reshape(N) for o in outs)


def _build_collapse(interpret=False, num_cores=None, num_subcores=None):
    if num_cores is None:
        mesh = plsc.VectorSubcoreMesh(core_axis_name="c", subcore_axis_name="s")
    else:
        mesh = plsc.VectorSubcoreMesh(core_axis_name="c", subcore_axis_name="s",
                                      num_cores=num_cores,
                                      num_subcores=num_subcores)
    nc, ns = mesh.num_cores, mesh.num_subcores
    nw = nc * ns
    rpt = N // nw

    @functools.partial(
        pl.kernel,
        out_type=(jax.ShapeDtypeStruct((N,), jnp.float32),
                  jax.ShapeDtypeStruct((N,), jnp.float32)),
        mesh=mesh,
        interpret=interpret,
        compiler_params=pltpu.CompilerParams(needs_layout_passes=False),
        scratch_types=[
            pltpu.VMEM((N,), jnp.float32),
            pltpu.VMEM((N,), jnp.float32),
            pltpu.VMEM((rpt,), jnp.float32),
            pltpu.VMEM((rpt,), jnp.int32),
            pltpu.VMEM((BATCH, N), jnp.float32),
            pltpu.VMEM((BATCH, N), jnp.float32),
            pltpu.VMEM((rpt,), jnp.float32),
            pltpu.VMEM((rpt,), jnp.float32),
            pltpu.SemaphoreType.DMA,
            pltpu.SemaphoreType.DMA,
        ],
    )
    def collapse(wr_hbm, wi_hbm, lr_hbm, li_hbm, amt_hbm,
                 outr_hbm, outi_hbm,
                 lr_v, li_v, amt_v, idx_v, rowr_v, rowi_v,
                 outr_v, outi_v, semr, semi):
        wid = lax.axis_index("s") * nc + lax.axis_index("c")
        row0 = wid * rpt

        pltpu.sync_copy(lr_hbm, lr_v)
        pltpu.sync_copy(li_hbm, li_v)
        pltpu.sync_copy(amt_hbm.at[pl.ds(row0, rpt)], amt_v)
        pltpu.sync_copy(lr_hbm.at[pl.ds(row0, rpt)], outr_v)
        pltpu.sync_copy(li_hbm.at[pl.ds(row0, rpt)], outi_v)

        iota = lax.broadcasted_iota(jnp.int32, (LANES,), 0)
        row0_v = jnp.full((LANES,), row0, jnp.int32)

        for k in range(rpt // LANES):
            idx_v[pl.ds(k * LANES, LANES)] = row0_v

        cnt = jnp.int32(0)
        for k in range(rpt // LANES):
            m = amt_v[pl.ds(k * LANES, LANES)] >= 0.0
            mi = m.astype(jnp.int32)
            pos = cnt + jnp.cumsum(mi) - 1
            plsc.store_scatter(idx_v, [pos], row0_v + (k * LANES) + iota,
                               mask=m)
            cnt = cnt + jnp.sum(mi)

        nb = (cnt + (BATCH - 1)) // BATCH
        zeros = jnp.zeros((LANES,), jnp.float32)
        cnt_v = jnp.full((LANES,), cnt, jnp.int32)

        @pl.when(nb > 0)
        def _():
            pltpu.async_copy(wr_hbm.at[idx_v.at[pl.ds(0, BATCH)]], rowr_v,
                             semr)

        def batch_body(b, carry):
            sl = idx_v.at[pl.ds(b * BATCH, BATCH)]
            pltpu.make_async_copy(wr_hbm.at[sl], rowr_v, semr).wait()
            pltpu.async_copy(wi_hbm.at[sl], rowi_v, semi)

            def chunk_a(c, accs):
                fr_t, fi_t = accs
                base = c * LANES
                lr_c = lr_v[pl.ds(base, LANES)]
                li_c = li_v[pl.ds(base, LANES)]
                nfr, nfi = [], []
                for r in range(BATCH):
                    w_r = rowr_v[r, pl.ds(base, LANES)]
                    nfr.append(fr_t[r] + w_r * lr_c)
                    nfi.append(fi_t[r] + w_r * li_c)
                return (tuple(nfr), tuple(nfi))

            init = (tuple([zeros] * BATCH), tuple([zeros] * BATCH))
            fr_t, fi_t = lax.fori_loop(0, NCHUNK, chunk_a, init)

            pltpu.make_async_copy(wi_hbm.at[sl], rowi_v, semi).wait()

            @pl.when(b + 1 < nb)
            def _():
                pltpu.async_copy(
                    wr_hbm.at[idx_v.at[pl.ds((b + 1) * BATCH, BATCH)]],
                    rowr_v, semr)

            def chunk_b(c, accs):
                fr_t, fi_t = accs
                base = c * LANES
                lr_c = lr_v[pl.ds(base, LANES)]
                li_c = li_v[pl.ds(base, LANES)]
                nfr, nfi = [], []
                for r in range(BATCH):
                    w_i = rowi_v[r, pl.ds(base, LANES)]
                    nfr.append(fr_t[r] - w_i * li_c)
                    nfi.append(fi_t[r] + w_i * lr_c)
                return (tuple(nfr), tuple(nfi))

            fr_t, fi_t = lax.fori_loop(0, NCHUNK, chunk_b, (fr_t, fi_t))

            for r in range(BATCH):
                slot = b * BATCH + r
                slot_v = jnp.full((LANES,), slot, jnp.int32)
                gidx_v = plsc.load_gather(idx_v, [slot_v])
                lidx_v = gidx_v - row0_v
                lrv = plsc.load_gather(lr_v, [gidx_v])
                liv = plsc.load_gather(li_v, [gidx_v])
                fr_v = jnp.full((LANES,), jnp.sum(fr_t[r]))
                fi_v = jnp.full((LANES,), jnp.sum(fi_t[r]))
                new_r = lrv * jnp.maximum(fr_v, 0.0)
                new_i = liv * fi_v
                mk = (iota == 0) & (slot_v < cnt_v)
                plsc.store_scatter(outr_v, [lidx_v], new_r, mask=mk)
                plsc.store_scatter(outi_v, [lidx_v], new_i, mask=mk)
            return carry

        lax.fori_loop(0, nb, batch_body, jnp.int32(0))

        pltpu.sync_copy(outr_v, outr_hbm.at[pl.ds(row0, rpt)])
        pltpu.sync_copy(outi_v, outi_hbm.at[pl.ds(row0, rpt)])

    return collapse


def kernel(t_span, dt, A_real, A_imag, w_acc_real, w_acc_imag, theta,
           W_filter_real, W_filter_imag, L_real_init, L_imag_init):
    num_steps = t_span.shape[0] - 1
    dtf = jnp.asarray(dt, jnp.float32)
    collapse = _build_collapse()
    Lr, Li = L_real_init, L_imag_init
    reals, imags = [], []
    for _ in range(num_steps):
        lr1, li1, amt = _prep(dtf, A_real, A_imag, w_acc_real, w_acc_imag,
                              theta, Lr, Li)
        Lr, Li = collapse(W_filter_real, W_filter_imag, lr1, li1, amt)
        reals.append(Lr)
        imags.append(Li)
    return jnp.stack(reals), jnp.stack(imags)


# speedup vs baseline: 2.0899x; 1.0504x over previous
import functools

import jax
import jax.numpy as jnp
from jax import lax
from jax.experimental import pallas as pl
from jax.experimental.pallas import tpu as pltpu
from jax.experimental.pallas import tpu_sc as plsc

N = 4096
LANES = 16
BATCH = 8
NCHUNK = N // LANES


def _prep_body(dt_ref, ar, ai, wr, wi, th, lr0, li0, olr, oli, oamt):
    dtf = dt_ref[0, 0]
    er = jnp.exp(ar[...] * dtf)
    exp_r = er * jnp.cos(ai[...] * dtf)
    exp_i = er * jnp.sin(ai[...] * dtf)
    lr = lr0[...] * exp_r - li0[...] * exp_i
    li = lr0[...] * exp_i + li0[...] * exp_r
    olr[...] = lr
    oli[...] = li
    oamt[...] = wr[...] * lr + wi[...] * li - th[...]


def _prep(dtf, ar, ai, wr, wi, th, lr0, li0, interpret=False):
    shp = (N // 128, 128)
    vspec = pl.BlockSpec(memory_space=pltpu.VMEM)
    outs = pl.pallas_call(
        _prep_body,
        out_shape=[jax.ShapeDtypeStruct(shp, jnp.float32)] * 3,
        in_specs=[pl.BlockSpec(memory_space=pltpu.SMEM)] + [vspec] * 7,
        out_specs=[vspec] * 3,
        interpret=interpret,
    )(dtf.reshape(1, 1), ar.reshape(shp), ai.reshape(shp), wr.reshape(shp),
      wi.reshape(shp), th.reshape(shp), lr0.reshape(shp), li0.reshape(shp))
    return tuple(o.reshape(N) for o in outs)


def _build_collapse(interpret=False, num_cores=None, num_subcores=None):
    if num_cores is None:
        mesh = plsc.VectorSubcoreMesh(core_axis_name="c", subcore_axis_name="s")
    else:
        mesh = plsc.VectorSubcoreMesh(core_axis_name="c", subcore_axis_name="s",
                                      num_cores=num_cores,
                                      num_subcores=num_subcores)
    nc, ns = mesh.num_cores, mesh.num_subcores
    nw = nc * ns
    rpt = N // nw

    @functools.partial(
        pl.kernel,
        out_type=(jax.ShapeDtypeStruct((N,), jnp.float32),
                  jax.ShapeDtypeStruct((N,), jnp.float32)),
        mesh=mesh,
        interpret=interpret,
        compiler_params=pltpu.CompilerParams(needs_layout_passes=False),
        scratch_types=[
            pltpu.VMEM((N,), jnp.float32),
            pltpu.VMEM((N,), jnp.float32),
            pltpu.VMEM((rpt,), jnp.float32),
            pltpu.VMEM((rpt,), jnp.int32),
            pltpu.VMEM((BATCH, N), jnp.float32),
            pltpu.VMEM((BATCH, N), jnp.float32),
            pltpu.VMEM((rpt,), jnp.float32),
            pltpu.VMEM((rpt,), jnp.float32),
            pltpu.SemaphoreType.DMA,
            pltpu.SemaphoreType.DMA,
        ],
    )
    def collapse(wr_hbm, wi_hbm, lr_hbm, li_hbm, amt_hbm,
                 outr_hbm, outi_hbm,
                 lr_v, li_v, amt_v, idx_v, rowr_v, rowi_v,
                 outr_v, outi_v, semr, semi):
        wid = lax.axis_index("s") * nc + lax.axis_index("c")
        row0 = wid * rpt

        pltpu.sync_copy(lr_hbm, lr_v)
        pltpu.sync_copy(li_hbm, li_v)
        pltpu.sync_copy(amt_hbm.at[pl.ds(row0, rpt)], amt_v)
        pltpu.sync_copy(lr_hbm.at[pl.ds(row0, rpt)], outr_v)
        pltpu.sync_copy(li_hbm.at[pl.ds(row0, rpt)], outi_v)

        iota = lax.broadcasted_iota(jnp.int32, (LANES,), 0)
        row0_v = jnp.full((LANES,), row0, jnp.int32)

        for k in range(rpt // LANES):
            idx_v[pl.ds(k * LANES, LANES)] = row0_v

        cnt = jnp.int32(0)
        for k in range(rpt // LANES):
            m = amt_v[pl.ds(k * LANES, LANES)] >= 0.0
            mi = m.astype(jnp.int32)
            pos = cnt + jnp.cumsum(mi) - 1
            plsc.store_scatter(idx_v, [pos], row0_v + (k * LANES) + iota,
                               mask=m)
            cnt = cnt + jnp.sum(mi)

        nb = (cnt + (BATCH - 1)) // BATCH
        zeros = jnp.zeros((LANES,), jnp.float32)
        cnt_v = jnp.full((LANES,), cnt, jnp.int32)

        @pl.when(nb > 0)
        def _():
            pltpu.async_copy(wr_hbm.at[idx_v.at[pl.ds(0, BATCH)]], rowr_v,
                             semr)

        def batch_body(b, carry):
            sl = idx_v.at[pl.ds(b * BATCH, BATCH)]
            pltpu.make_async_copy(wr_hbm.at[sl], rowr_v, semr).wait()
            pltpu.async_copy(wi_hbm.at[sl], rowi_v, semi)

            init = (tuple([zeros] * BATCH), tuple([zeros] * BATCH))

            @plsc.parallel_loop(0, NCHUNK, unroll=4, carry=init)
            def acc_a(c, accs):
                fr_t, fi_t = accs
                base = c * LANES
                lr_c = lr_v[pl.ds(base, LANES)]
                li_c = li_v[pl.ds(base, LANES)]
                nfr, nfi = [], []
                for r in range(BATCH):
                    w_r = rowr_v[r, pl.ds(base, LANES)]
                    nfr.append(fr_t[r] + w_r * lr_c)
                    nfi.append(fi_t[r] + w_r * li_c)
                return (tuple(nfr), tuple(nfi))

            fr_t, fi_t = acc_a

            pltpu.make_async_copy(wi_hbm.at[sl], rowi_v, semi).wait()

            @pl.when(b + 1 < nb)
            def _():
                pltpu.async_copy(
                    wr_hbm.at[idx_v.at[pl.ds((b + 1) * BATCH, BATCH)]],
                    rowr_v, semr)

            @plsc.parallel_loop(0, NCHUNK, unroll=4, carry=(fr_t, fi_t))
            def acc_b(c, accs):
                fr_t, fi_t = accs
                base = c * LANES
                lr_c = lr_v[pl.ds(base, LANES)]
                li_c = li_v[pl.ds(base, LANES)]
                nfr, nfi = [], []
                for r in range(BATCH):
                    w_i = rowi_v[r, pl.ds(base, LANES)]
                    nfr.append(fr_t[r] - w_i * li_c)
                    nfi.append(fi_t[r] + w_i * lr_c)
                return (tuple(nfr), tuple(nfi))

            fr_t, fi_t = acc_b

            for r in range(BATCH):
                slot = b * BATCH + r
                slot_v = jnp.full((LANES,), slot, jnp.int32)
                gidx_v = plsc.load_gather(idx_v, [slot_v])
                lidx_v = gidx_v - row0_v
                lrv = plsc.load_gather(lr_v, [gidx_v])
                liv = plsc.load_gather(li_v, [gidx_v])
                fr_v = jnp.full((LANES,), jnp.sum(fr_t[r]))
                fi_v = jnp.full((LANES,), jnp.sum(fi_t[r]))
                new_r = lrv * jnp.maximum(fr_v, 0.0)
                new_i = liv * fi_v
                mk = (iota == 0) & (slot_v < cnt_v)
                plsc.store_scatter(outr_v, [lidx_v], new_r, mask=mk)
                plsc.store_scatter(outi_v, [lidx_v], new_i, mask=mk)
            return carry

        lax.fori_loop(0, nb, batch_body, jnp.int32(0))

        pltpu.sync_copy(outr_v, outr_hbm.at[pl.ds(row0, rpt)])
        pltpu.sync_copy(outi_v, outi_hbm.at[pl.ds(row0, rpt)])

    return collapse


def kernel(t_span, dt, A_real, A_imag, w_acc_real, w_acc_imag, theta,
           W_filter_real, W_filter_imag, L_real_init, L_imag_init):
    num_steps = t_span.shape[0] - 1
    dtf = jnp.asarray(dt, jnp.float32)
    collapse = _build_collapse()
    Lr, Li = L_real_init, L_imag_init
    reals, imags = [], []
    for _ in range(num_steps):
        lr1, li1, amt = _prep(dtf, A_real, A_imag, w_acc_real, w_acc_imag,
                              theta, Lr, Li)
        Lr, Li = collapse(W_filter_real, W_filter_imag, lr1, li1, amt)
        reals.append(Lr)
        imags.append(Li)
    return jnp.stack(reals), jnp.stack(imags)
